# baseline (speedup 1.0000x reference)
import functools

import jax
import jax.numpy as jnp
from jax.experimental import pallas as pl
from jax.experimental.pallas import tpu as pltpu

_BR = 8
_BC = 2048


def _nll_kernel(bins_ref, targets_ref, logits_ref, out_ref):
    b = bins_ref[0, :]
    n = b.shape[0] - 1
    lo = jnp.where(jax.lax.iota(jnp.int32, n) == 0, -jnp.inf, b[:n])
    hi = jnp.where(jax.lax.iota(jnp.int32, n) == n - 1, jnp.inf, b[1:])
    lw = jnp.log(b[1:] - b[:n])

    t = targets_ref[...]
    x = logits_ref[...]

    tm = t[:, :, None]
    mask = (tm > lo[None, None, :]) & (tm <= hi[None, None, :])

    s = jnp.sum(jnp.exp(x), axis=-1)
    g = jnp.sum(jnp.where(mask, x - lw[None, None, :], 0.0), axis=-1)
    out_ref[...] = jnp.log(s) - g


@jax.jit
def kernel(targets, logits, bins):
    R, C = targets.shape
    nb = bins.shape[0]
    grid = (R // _BR, C // _BC)
    return pl.pallas_call(
        _nll_kernel,
        grid=grid,
        in_specs=[
            pl.BlockSpec((1, nb), lambda i, j: (0, 0)),
            pl.BlockSpec((_BR, _BC), lambda i, j: (i, j)),
            pl.BlockSpec((_BR, _BC, 32), lambda i, j: (i, j, 0)),
        ],
        out_specs=pl.BlockSpec((_BR, _BC), lambda i, j: (i, j)),
        out_shape=jax.ShapeDtypeStruct((R, C), jnp.float32),
    )(bins.reshape(1, nb), targets, logits)
